# Initial kernel scaffold; baseline (speedup 1.0000x reference)
import jax
import jax.numpy as jnp
from jax.experimental import pallas as pl


def kernel(x, edge0_src, edge0_dst, edge1_src, edge1_dst, W_proj, b_proj, Wq1, bq1, Ww1, bw1, Wq2, bq2, Ww2, bw2):
    raise NotImplementedError("write your pallas kernel here")


# speedup vs baseline: 4.8308x; 4.8308x over previous
import functools

import jax
import jax.numpy as jnp
from jax import lax
from jax.experimental import pallas as pl
from jax.experimental.pallas import tpu as pltpu
from jax.experimental.pallas import tpu_sc as plsc

N0, N1, N2 = 10000, 4096, 1024
E0, E1 = 320000, 32768
D = 128

NC, NS = 2, 16
NW = NC * NS


def _make_edge_agg(n_edges, n_dst, chunk):
    epw = n_edges // NW
    nchunks = epw // chunk
    assert epw % chunk == 0 and chunk % 8 == 0 and chunk <= 128
    zr = n_dst // NS
    mesh = plsc.VectorSubcoreMesh(core_axis_name="c", subcore_axis_name="s")

    @functools.partial(
        pl.kernel,
        out_type=(
            jax.ShapeDtypeStruct((NC, n_dst, D), jnp.float32),
            jax.ShapeDtypeStruct((NC, n_dst, D), jnp.float32),
        ),
        mesh=mesh,
        scratch_types=[
            pltpu.VMEM((chunk,), jnp.int32),
            pltpu.VMEM((chunk,), jnp.int32),
            pltpu.VMEM((chunk, D), jnp.float32),
            pltpu.VMEM((chunk, D), jnp.float32),
            pltpu.VMEM_SHARED((n_dst, D), jnp.float32),
            pltpu.VMEM_SHARED((n_dst, D), jnp.float32),
            pltpu.SemaphoreType.DMA,
        ],
    )
    def edge_agg(m_hbm, src_hbm, dst_hbm, zagg_hbm, ones_hbm, agg_out,
                 cnt_out, src_v, dst_v, buf, ones, agg_sh, cnt_sh, sem):
        cid = lax.axis_index("c")
        sid = lax.axis_index("s")
        wid = cid * NS + sid

        pltpu.sync_copy(zagg_hbm, agg_sh.at[pl.ds(sid * zr, zr)])
        pltpu.sync_copy(zagg_hbm, cnt_sh.at[pl.ds(sid * zr, zr)])
        pltpu.sync_copy(ones_hbm, ones)
        plsc.subcore_barrier()

        base = wid * epw

        @pl.loop(0, nchunks)
        def _(c):
            off = base + c * chunk
            pltpu.sync_copy(src_hbm.at[pl.ds(off, chunk)], src_v)
            pltpu.sync_copy(dst_hbm.at[pl.ds(off, chunk)], dst_v)
            pltpu.async_copy(m_hbm.at[src_v], buf, sem).wait()
            pltpu.sync_copy(buf, agg_sh.at[dst_v], add=True)
            pltpu.sync_copy(ones, cnt_sh.at[dst_v], add=True)

        plsc.subcore_barrier()
        pltpu.sync_copy(agg_sh.at[pl.ds(sid * zr, zr)],
                        agg_out.at[cid, pl.ds(sid * zr, zr)])
        pltpu.sync_copy(cnt_sh.at[pl.ds(sid * zr, zr)],
                        cnt_out.at[cid, pl.ds(sid * zr, zr)])

    return edge_agg


def _dot(a, b):
    return jnp.dot(a, b, preferred_element_type=jnp.float32,
                   precision=lax.Precision.HIGHEST)


def _stage_a_body(x_ref, wp_ref, bp_ref, wq_ref, bq_ref, h_ref, m_ref):
    xb = x_ref[...]
    hb = _dot(xb, wp_ref[...]) + bp_ref[...]
    h_ref[...] = hb
    m_ref[...] = jnp.maximum(_dot(hb, wq_ref[...]) + bq_ref[...], 0.0)


def _stage_a(x, W_proj, b_proj, Wq1, bq1):
    n = x.shape[0]
    blk = 1000
    grid = n // blk
    full = lambda i: (0, 0)
    return pl.pallas_call(
        _stage_a_body,
        grid=(grid,),
        in_specs=[
            pl.BlockSpec((blk, D), lambda i: (i, 0)),
            pl.BlockSpec((D, D), full),
            pl.BlockSpec((1, D), full),
            pl.BlockSpec((D, D), full),
            pl.BlockSpec((1, D), full),
        ],
        out_specs=[
            pl.BlockSpec((blk, D), lambda i: (i, 0)),
            pl.BlockSpec((blk, D), lambda i: (i, 0)),
        ],
        out_shape=[
            jax.ShapeDtypeStruct((n, D), jnp.float32),
            jax.ShapeDtypeStruct((n, D), jnp.float32),
        ],
    )(x, W_proj, b_proj.reshape(1, D), Wq1, bq1.reshape(1, D))


def _mean_from_parts(agg_ref, cnt_ref):
    p = agg_ref[...]
    c = cnt_ref[...]
    cnt = c[0, :, 0] + c[1, :, 0]
    return (p[0] + p[1]) / jnp.maximum(cnt, 1.0)[:, None]


def _l2norm(z):
    zn = jnp.sqrt(jnp.sum(z * z, axis=1, keepdims=True))
    return z / jnp.maximum(zn, 1e-12)


def _stage_c_body(h_ref, agg_ref, cnt_ref, wa_ref, wb_ref, bw_ref,
                  wq_ref, bq_ref, h1_ref, m2_ref):
    mean = _mean_from_parts(agg_ref, cnt_ref)
    z = _dot(h_ref[...], wa_ref[...]) + _dot(mean, wb_ref[...]) + bw_ref[...]
    h1 = _l2norm(jnp.maximum(z, 0.0))
    h1_ref[...] = h1
    m2_ref[...] = jnp.maximum(_dot(h1, wq_ref[...]) + bq_ref[...], 0.0)


def _stage_c(h4096, agg_parts, cnt_parts, Ww1, bw1, Wq2, bq2):
    blk = 512
    grid = N1 // blk
    full = lambda i: (0, 0)
    return pl.pallas_call(
        _stage_c_body,
        grid=(grid,),
        in_specs=[
            pl.BlockSpec((blk, D), lambda i: (i, 0)),
            pl.BlockSpec((NC, blk, D), lambda i: (0, i, 0)),
            pl.BlockSpec((NC, blk, D), lambda i: (0, i, 0)),
            pl.BlockSpec((D, D), full),
            pl.BlockSpec((D, D), full),
            pl.BlockSpec((1, D), full),
            pl.BlockSpec((D, D), full),
            pl.BlockSpec((1, D), full),
        ],
        out_specs=[
            pl.BlockSpec((blk, D), lambda i: (i, 0)),
            pl.BlockSpec((blk, D), lambda i: (i, 0)),
        ],
        out_shape=[
            jax.ShapeDtypeStruct((N1, D), jnp.float32),
            jax.ShapeDtypeStruct((N1, D), jnp.float32),
        ],
    )(h4096, agg_parts, cnt_parts, Ww1[:D], Ww1[D:], bw1.reshape(1, D),
      Wq2, bq2.reshape(1, D))


def _stage_d_body(h_ref, h1_ref, agg_ref, cnt_ref, wa_ref, wb_ref, bw_ref,
                  out_ref):
    mean = _mean_from_parts(agg_ref, cnt_ref)
    z = _dot(h1_ref[...], wa_ref[...]) + _dot(mean, wb_ref[...]) + bw_ref[...]
    out_ref[...] = h_ref[...] + _l2norm(jnp.maximum(z, 0.0))


def _stage_d(h1024, h1_1024, agg_parts, cnt_parts, Ww2, bw2):
    blk = 512
    grid = N2 // blk
    full = lambda i: (0, 0)
    return pl.pallas_call(
        _stage_d_body,
        grid=(grid,),
        in_specs=[
            pl.BlockSpec((blk, D), lambda i: (i, 0)),
            pl.BlockSpec((blk, D), lambda i: (i, 0)),
            pl.BlockSpec((NC, blk, D), lambda i: (0, i, 0)),
            pl.BlockSpec((NC, blk, D), lambda i: (0, i, 0)),
            pl.BlockSpec((D, D), full),
            pl.BlockSpec((D, D), full),
            pl.BlockSpec((1, D), full),
        ],
        out_specs=pl.BlockSpec((blk, D), lambda i: (i, 0)),
        out_shape=jax.ShapeDtypeStruct((N2, D), jnp.float32),
    )(h1024, h1_1024, agg_parts, cnt_parts, Ww2[:D], Ww2[D:],
      bw2.reshape(1, D))


_edge_agg0 = _make_edge_agg(E0, N1, 80)
_edge_agg1 = _make_edge_agg(E1, N2, 128)


def kernel(x, edge0_src, edge0_dst, edge1_src, edge1_dst,
           W_proj, b_proj, Wq1, bq1, Ww1, bw1, Wq2, bq2, Ww2, bw2):
    h, m1 = _stage_a(x, W_proj, b_proj, Wq1, bq1)
    zagg0 = jnp.zeros((N1 // NS, D), jnp.float32)
    ones0 = jnp.ones((80, D), jnp.float32)
    agg1, cnt1 = _edge_agg0(m1, edge0_src, edge0_dst, zagg0, ones0)
    h1, m2 = _stage_c(h[:N1], agg1, cnt1, Ww1, bw1, Wq2, bq2)
    zagg1 = jnp.zeros((N2 // NS, D), jnp.float32)
    ones1 = jnp.ones((128, D), jnp.float32)
    agg2, cnt2 = _edge_agg1(m2, edge1_src, edge1_dst, zagg1, ones1)
    return _stage_d(h[:N2], h1[:N2], agg2, cnt2, Ww2, bw2)


# speedup vs baseline: 7.6685x; 1.5874x over previous
import functools

import jax
import jax.numpy as jnp
from jax import lax
from jax.experimental import pallas as pl
from jax.experimental.pallas import tpu as pltpu
from jax.experimental.pallas import tpu_sc as plsc

N0, N1, N2 = 10000, 4096, 1024
E0, E1 = 320000, 32768
D = 128

NC, NS = 2, 16
NW = NC * NS


def _make_edge_agg(n_edges, n_dst, chunk):
    epw = n_edges // NW
    nchunks = epw // chunk
    assert epw % chunk == 0 and chunk % 8 == 0 and chunk <= 128
    zr = n_dst // NS
    mesh = plsc.VectorSubcoreMesh(core_axis_name="c", subcore_axis_name="s")

    @functools.partial(
        pl.kernel,
        out_type=(
            jax.ShapeDtypeStruct((NC, n_dst, D), jnp.float32),
            jax.ShapeDtypeStruct((NC, n_dst, D), jnp.float32),
        ),
        mesh=mesh,
        scratch_types=[
            pltpu.VMEM((chunk,), jnp.int32),
            pltpu.VMEM((chunk,), jnp.int32),
            pltpu.VMEM((chunk,), jnp.int32),
            pltpu.VMEM((chunk,), jnp.int32),
            pltpu.VMEM((chunk, D), jnp.float32),
            pltpu.VMEM((chunk, D), jnp.float32),
            pltpu.VMEM((chunk, D), jnp.float32),
            pltpu.VMEM_SHARED((n_dst, D), jnp.float32),
            pltpu.VMEM_SHARED((n_dst, D), jnp.float32),
            pltpu.SemaphoreType.DMA,
            pltpu.SemaphoreType.DMA,
            pltpu.SemaphoreType.DMA,
            pltpu.SemaphoreType.DMA,
            pltpu.SemaphoreType.DMA,
            pltpu.SemaphoreType.DMA,
        ],
    )
    def edge_agg(m_hbm, src_hbm, dst_hbm, zagg_hbm, ones_hbm, agg_out,
                 cnt_out, src_v0, dst_v0, src_v1, dst_v1, buf0, buf1, ones,
                 agg_sh, cnt_sh, gsem0, gsem1, asem0, asem1, osem0, osem1):
        cid = lax.axis_index("c")
        sid = lax.axis_index("s")
        wid = cid * NS + sid

        pltpu.sync_copy(zagg_hbm, agg_sh.at[pl.ds(sid * zr, zr)])
        pltpu.sync_copy(zagg_hbm, cnt_sh.at[pl.ds(sid * zr, zr)])
        pltpu.sync_copy(ones_hbm, ones)
        plsc.subcore_barrier()

        base = wid * epw

        def start(c, sv, dv, bf, gsem, osem):
            off = base + c * chunk
            pltpu.sync_copy(src_hbm.at[pl.ds(off, chunk)], sv)
            pltpu.sync_copy(dst_hbm.at[pl.ds(off, chunk)], dv)
            pltpu.async_copy(m_hbm.at[sv], bf, gsem)
            pltpu.async_copy(ones, cnt_sh.at[dv], osem, add=True)

        start(0, src_v0, dst_v0, buf0, gsem0, osem0)

        @pl.loop(0, (nchunks + 1) // 2)
        def _(i):
            c = 2 * i

            @pl.when(c + 1 < nchunks)
            def _():
                start(c + 1, src_v1, dst_v1, buf1, gsem1, osem1)

            pltpu.make_async_copy(m_hbm.at[src_v0], buf0, gsem0).wait()
            pltpu.async_copy(buf0, agg_sh.at[dst_v0], asem0, add=True)

            @pl.when(c + 1 < nchunks)
            def _():
                pltpu.make_async_copy(m_hbm.at[src_v1], buf1, gsem1).wait()
                pltpu.async_copy(buf1, agg_sh.at[dst_v1], asem1, add=True)

            pltpu.make_async_copy(buf0, agg_sh.at[dst_v0], asem0).wait()
            pltpu.make_async_copy(ones, cnt_sh.at[dst_v0], osem0).wait()

            @pl.when(c + 2 < nchunks)
            def _():
                start(c + 2, src_v0, dst_v0, buf0, gsem0, osem0)

            @pl.when(c + 1 < nchunks)
            def _():
                pltpu.make_async_copy(buf1, agg_sh.at[dst_v1], asem1).wait()
                pltpu.make_async_copy(ones, cnt_sh.at[dst_v1], osem1).wait()

        plsc.subcore_barrier()
        pltpu.sync_copy(agg_sh.at[pl.ds(sid * zr, zr)],
                        agg_out.at[cid, pl.ds(sid * zr, zr)])
        pltpu.sync_copy(cnt_sh.at[pl.ds(sid * zr, zr)],
                        cnt_out.at[cid, pl.ds(sid * zr, zr)])

    return edge_agg


def _dot(a, b):
    return jnp.dot(a, b, preferred_element_type=jnp.float32,
                   precision=lax.Precision.HIGHEST)


def _stage_a_body(x_ref, wp_ref, bp_ref, wq_ref, bq_ref, h_ref, m_ref):
    xb = x_ref[...]
    hb = _dot(xb, wp_ref[...]) + bp_ref[...]
    h_ref[...] = hb
    m_ref[...] = jnp.maximum(_dot(hb, wq_ref[...]) + bq_ref[...], 0.0)


def _stage_a(x, W_proj, b_proj, Wq1, bq1):
    n = x.shape[0]
    blk = 1000
    grid = n // blk
    full = lambda i: (0, 0)
    return pl.pallas_call(
        _stage_a_body,
        grid=(grid,),
        in_specs=[
            pl.BlockSpec((blk, D), lambda i: (i, 0)),
            pl.BlockSpec((D, D), full),
            pl.BlockSpec((1, D), full),
            pl.BlockSpec((D, D), full),
            pl.BlockSpec((1, D), full),
        ],
        out_specs=[
            pl.BlockSpec((blk, D), lambda i: (i, 0)),
            pl.BlockSpec((blk, D), lambda i: (i, 0)),
        ],
        out_shape=[
            jax.ShapeDtypeStruct((n, D), jnp.float32),
            jax.ShapeDtypeStruct((n, D), jnp.float32),
        ],
    )(x, W_proj, b_proj.reshape(1, D), Wq1, bq1.reshape(1, D))


def _mean_from_parts(agg_ref, cnt_ref):
    p = agg_ref[...]
    c = cnt_ref[...]
    cnt = c[0, :, 0] + c[1, :, 0]
    return (p[0] + p[1]) / jnp.maximum(cnt, 1.0)[:, None]


def _l2norm(z):
    zn = jnp.sqrt(jnp.sum(z * z, axis=1, keepdims=True))
    return z / jnp.maximum(zn, 1e-12)


def _stage_c_body(h_ref, agg_ref, cnt_ref, wa_ref, wb_ref, bw_ref,
                  wq_ref, bq_ref, h1_ref, m2_ref):
    mean = _mean_from_parts(agg_ref, cnt_ref)
    z = _dot(h_ref[...], wa_ref[...]) + _dot(mean, wb_ref[...]) + bw_ref[...]
    h1 = _l2norm(jnp.maximum(z, 0.0))
    h1_ref[...] = h1
    m2_ref[...] = jnp.maximum(_dot(h1, wq_ref[...]) + bq_ref[...], 0.0)


def _stage_c(h4096, agg_parts, cnt_parts, Ww1, bw1, Wq2, bq2):
    blk = 512
    grid = N1 // blk
    full = lambda i: (0, 0)
    return pl.pallas_call(
        _stage_c_body,
        grid=(grid,),
        in_specs=[
            pl.BlockSpec((blk, D), lambda i: (i, 0)),
            pl.BlockSpec((NC, blk, D), lambda i: (0, i, 0)),
            pl.BlockSpec((NC, blk, D), lambda i: (0, i, 0)),
            pl.BlockSpec((D, D), full),
            pl.BlockSpec((D, D), full),
            pl.BlockSpec((1, D), full),
            pl.BlockSpec((D, D), full),
            pl.BlockSpec((1, D), full),
        ],
        out_specs=[
            pl.BlockSpec((blk, D), lambda i: (i, 0)),
            pl.BlockSpec((blk, D), lambda i: (i, 0)),
        ],
        out_shape=[
            jax.ShapeDtypeStruct((N1, D), jnp.float32),
            jax.ShapeDtypeStruct((N1, D), jnp.float32),
        ],
    )(h4096, agg_parts, cnt_parts, Ww1[:D], Ww1[D:], bw1.reshape(1, D),
      Wq2, bq2.reshape(1, D))


def _stage_d_body(h_ref, h1_ref, agg_ref, cnt_ref, wa_ref, wb_ref, bw_ref,
                  out_ref):
    mean = _mean_from_parts(agg_ref, cnt_ref)
    z = _dot(h1_ref[...], wa_ref[...]) + _dot(mean, wb_ref[...]) + bw_ref[...]
    out_ref[...] = h_ref[...] + _l2norm(jnp.maximum(z, 0.0))


def _stage_d(h1024, h1_1024, agg_parts, cnt_parts, Ww2, bw2):
    blk = 512
    grid = N2 // blk
    full = lambda i: (0, 0)
    return pl.pallas_call(
        _stage_d_body,
        grid=(grid,),
        in_specs=[
            pl.BlockSpec((blk, D), lambda i: (i, 0)),
            pl.BlockSpec((blk, D), lambda i: (i, 0)),
            pl.BlockSpec((NC, blk, D), lambda i: (0, i, 0)),
            pl.BlockSpec((NC, blk, D), lambda i: (0, i, 0)),
            pl.BlockSpec((D, D), full),
            pl.BlockSpec((D, D), full),
            pl.BlockSpec((1, D), full),
        ],
        out_specs=pl.BlockSpec((blk, D), lambda i: (i, 0)),
        out_shape=jax.ShapeDtypeStruct((N2, D), jnp.float32),
    )(h1024, h1_1024, agg_parts, cnt_parts, Ww2[:D], Ww2[D:],
      bw2.reshape(1, D))


_edge_agg0 = _make_edge_agg(E0, N1, 80)
_edge_agg1 = _make_edge_agg(E1, N2, 128)


def kernel(x, edge0_src, edge0_dst, edge1_src, edge1_dst,
           W_proj, b_proj, Wq1, bq1, Ww1, bw1, Wq2, bq2, Ww2, bw2):
    h, m1 = _stage_a(x, W_proj, b_proj, Wq1, bq1)
    zagg0 = jnp.zeros((N1 // NS, D), jnp.float32)
    ones0 = jnp.ones((80, D), jnp.float32)
    agg1, cnt1 = _edge_agg0(m1, edge0_src, edge0_dst, zagg0, ones0)
    h1, m2 = _stage_c(h[:N1], agg1, cnt1, Ww1, bw1, Wq2, bq2)
    zagg1 = jnp.zeros((N2 // NS, D), jnp.float32)
    ones1 = jnp.ones((128, D), jnp.float32)
    agg2, cnt2 = _edge_agg1(m2, edge1_src, edge1_dst, zagg1, ones1)
    return _stage_d(h[:N2], h1[:N2], agg2, cnt2, Ww2, bw2)


# speedup vs baseline: 8.3298x; 1.0862x over previous
import functools

import jax
import jax.numpy as jnp
from jax import lax
from jax.experimental import pallas as pl
from jax.experimental.pallas import tpu as pltpu
from jax.experimental.pallas import tpu_sc as plsc

N0, N1, N2 = 10000, 4096, 1024
E0, E1 = 320000, 32768
D = 128

NC, NS = 2, 16
NW = NC * NS


def _make_edge_agg(n_edges, n_dst, chunk=128):
    cpt = n_edges // (chunk * NW)
    nrem = (n_edges - cpt * chunk * NW) // chunk
    assert n_edges == (cpt * NW + nrem) * chunk
    assert nrem <= NW and chunk % 8 == 0 and chunk <= 128
    zr = n_dst // NS
    mesh = plsc.VectorSubcoreMesh(core_axis_name="c", subcore_axis_name="s")

    @functools.partial(
        pl.kernel,
        out_type=(
            jax.ShapeDtypeStruct((NC, n_dst, D), jnp.float32),
            jax.ShapeDtypeStruct((NC, n_dst, D), jnp.float32),
        ),
        mesh=mesh,
        scratch_types=[
            pltpu.VMEM((2, chunk), jnp.int32),
            pltpu.VMEM((2, chunk), jnp.int32),
            pltpu.VMEM((chunk, D), jnp.float32),
            pltpu.VMEM((chunk, D), jnp.float32),
            pltpu.VMEM((chunk, D), jnp.float32),
            pltpu.VMEM_SHARED((n_dst, D), jnp.float32),
            pltpu.VMEM_SHARED((n_dst, D), jnp.float32),
            pltpu.SemaphoreType.DMA,
            pltpu.SemaphoreType.DMA,
            pltpu.SemaphoreType.DMA,
            pltpu.SemaphoreType.DMA,
            pltpu.SemaphoreType.DMA,
            pltpu.SemaphoreType.DMA,
        ],
    )
    def edge_agg(m_hbm, e_hbm, zagg_hbm, ones_hbm, agg_out, cnt_out,
                 ev0, ev1, buf0, buf1, ones,
                 agg_sh, cnt_sh, gsem0, gsem1, asem0, asem1, osem0, osem1):
        cid = lax.axis_index("c")
        sid = lax.axis_index("s")
        wid = cid * NS + sid

        pltpu.sync_copy(zagg_hbm, agg_sh.at[pl.ds(sid * zr, zr)])
        pltpu.sync_copy(zagg_hbm, cnt_sh.at[pl.ds(sid * zr, zr)])
        pltpu.sync_copy(ones_hbm, ones)
        plsc.subcore_barrier()

        base = wid * (cpt * chunk)

        def start(off, ev, bf, gsem, osem):
            pltpu.sync_copy(e_hbm.at[:, pl.ds(off, chunk)], ev)
            pltpu.async_copy(m_hbm.at[ev.at[0]], bf, gsem)
            pltpu.async_copy(ones, cnt_sh.at[ev.at[1]], osem, add=True)

        def finish_gather(ev, bf, gsem, asem):
            pltpu.make_async_copy(m_hbm.at[ev.at[0]], bf, gsem).wait()
            pltpu.async_copy(bf, agg_sh.at[ev.at[1]], asem, add=True)

        def drain(ev, bf, asem, osem):
            pltpu.make_async_copy(bf, agg_sh.at[ev.at[1]], asem).wait()
            pltpu.make_async_copy(ones, cnt_sh.at[ev.at[1]], osem).wait()

        start(base, ev0, buf0, gsem0, osem0)

        @pl.loop(0, (cpt + 1) // 2)
        def _(i):
            c = 2 * i

            @pl.when(c + 1 < cpt)
            def _():
                start(base + (c + 1) * chunk, ev1, buf1, gsem1, osem1)

            finish_gather(ev0, buf0, gsem0, asem0)

            @pl.when(c + 1 < cpt)
            def _():
                finish_gather(ev1, buf1, gsem1, asem1)

            drain(ev0, buf0, asem0, osem0)

            @pl.when(c + 2 < cpt)
            def _():
                start(base + (c + 2) * chunk, ev0, buf0, gsem0, osem0)

            @pl.when(c + 1 < cpt)
            def _():
                drain(ev1, buf1, asem1, osem1)

        if nrem:
            @pl.when(wid < nrem)
            def _():
                off = cpt * chunk * NW + wid * chunk
                start(off, ev0, buf0, gsem0, osem0)
                finish_gather(ev0, buf0, gsem0, asem0)
                drain(ev0, buf0, asem0, osem0)

        plsc.subcore_barrier()
        pltpu.sync_copy(agg_sh.at[pl.ds(sid * zr, zr)],
                        agg_out.at[cid, pl.ds(sid * zr, zr)])
        pltpu.sync_copy(cnt_sh.at[pl.ds(sid * zr, zr)],
                        cnt_out.at[cid, pl.ds(sid * zr, zr)])

    return edge_agg


def _dot(a, b):
    return jnp.dot(a, b, preferred_element_type=jnp.float32,
                   precision=lax.Precision.HIGHEST)


def _stage_a_body(x_ref, wp_ref, bp_ref, wq_ref, bq_ref, h_ref, m_ref):
    xb = x_ref[...]
    hb = _dot(xb, wp_ref[...]) + bp_ref[...]
    h_ref[...] = hb
    m_ref[...] = jnp.maximum(_dot(hb, wq_ref[...]) + bq_ref[...], 0.0)


def _stage_a(x, W_proj, b_proj, Wq1, bq1):
    n = x.shape[0]
    blk = 1000
    grid = n // blk
    full = lambda i: (0, 0)
    return pl.pallas_call(
        _stage_a_body,
        grid=(grid,),
        in_specs=[
            pl.BlockSpec((blk, D), lambda i: (i, 0)),
            pl.BlockSpec((D, D), full),
            pl.BlockSpec((1, D), full),
            pl.BlockSpec((D, D), full),
            pl.BlockSpec((1, D), full),
        ],
        out_specs=[
            pl.BlockSpec((blk, D), lambda i: (i, 0)),
            pl.BlockSpec((blk, D), lambda i: (i, 0)),
        ],
        out_shape=[
            jax.ShapeDtypeStruct((n, D), jnp.float32),
            jax.ShapeDtypeStruct((n, D), jnp.float32),
        ],
    )(x, W_proj, b_proj.reshape(1, D), Wq1, bq1.reshape(1, D))


def _mean_from_parts(agg_ref, cnt_ref):
    p = agg_ref[...]
    c = cnt_ref[...]
    cnt = c[0, :, 0] + c[1, :, 0]
    return (p[0] + p[1]) / jnp.maximum(cnt, 1.0)[:, None]


def _l2norm(z):
    zn = jnp.sqrt(jnp.sum(z * z, axis=1, keepdims=True))
    return z / jnp.maximum(zn, 1e-12)


def _stage_c_body(h_ref, agg_ref, cnt_ref, wa_ref, wb_ref, bw_ref,
                  wq_ref, bq_ref, h1_ref, m2_ref):
    mean = _mean_from_parts(agg_ref, cnt_ref)
    z = _dot(h_ref[...], wa_ref[...]) + _dot(mean, wb_ref[...]) + bw_ref[...]
    h1 = _l2norm(jnp.maximum(z, 0.0))
    h1_ref[...] = h1
    m2_ref[...] = jnp.maximum(_dot(h1, wq_ref[...]) + bq_ref[...], 0.0)


def _stage_c(h4096, agg_parts, cnt_parts, Ww1, bw1, Wq2, bq2):
    blk = 512
    grid = N1 // blk
    full = lambda i: (0, 0)
    return pl.pallas_call(
        _stage_c_body,
        grid=(grid,),
        in_specs=[
            pl.BlockSpec((blk, D), lambda i: (i, 0)),
            pl.BlockSpec((NC, blk, D), lambda i: (0, i, 0)),
            pl.BlockSpec((NC, blk, D), lambda i: (0, i, 0)),
            pl.BlockSpec((D, D), full),
            pl.BlockSpec((D, D), full),
            pl.BlockSpec((1, D), full),
            pl.BlockSpec((D, D), full),
            pl.BlockSpec((1, D), full),
        ],
        out_specs=[
            pl.BlockSpec((blk, D), lambda i: (i, 0)),
            pl.BlockSpec((blk, D), lambda i: (i, 0)),
        ],
        out_shape=[
            jax.ShapeDtypeStruct((N1, D), jnp.float32),
            jax.ShapeDtypeStruct((N1, D), jnp.float32),
        ],
    )(h4096, agg_parts, cnt_parts, Ww1[:D], Ww1[D:], bw1.reshape(1, D),
      Wq2, bq2.reshape(1, D))


def _stage_d_body(h_ref, h1_ref, agg_ref, cnt_ref, wa_ref, wb_ref, bw_ref,
                  out_ref):
    mean = _mean_from_parts(agg_ref, cnt_ref)
    z = _dot(h1_ref[...], wa_ref[...]) + _dot(mean, wb_ref[...]) + bw_ref[...]
    out_ref[...] = h_ref[...] + _l2norm(jnp.maximum(z, 0.0))


def _stage_d(h1024, h1_1024, agg_parts, cnt_parts, Ww2, bw2):
    blk = 512
    grid = N2 // blk
    full = lambda i: (0, 0)
    return pl.pallas_call(
        _stage_d_body,
        grid=(grid,),
        in_specs=[
            pl.BlockSpec((blk, D), lambda i: (i, 0)),
            pl.BlockSpec((blk, D), lambda i: (i, 0)),
            pl.BlockSpec((NC, blk, D), lambda i: (0, i, 0)),
            pl.BlockSpec((NC, blk, D), lambda i: (0, i, 0)),
            pl.BlockSpec((D, D), full),
            pl.BlockSpec((D, D), full),
            pl.BlockSpec((1, D), full),
        ],
        out_specs=pl.BlockSpec((blk, D), lambda i: (i, 0)),
        out_shape=jax.ShapeDtypeStruct((N2, D), jnp.float32),
    )(h1024, h1_1024, agg_parts, cnt_parts, Ww2[:D], Ww2[D:],
      bw2.reshape(1, D))


_edge_agg0 = _make_edge_agg(E0, N1)
_edge_agg1 = _make_edge_agg(E1, N2)


def kernel(x, edge0_src, edge0_dst, edge1_src, edge1_dst,
           W_proj, b_proj, Wq1, bq1, Ww1, bw1, Wq2, bq2, Ww2, bw2):
    h, m1 = _stage_a(x, W_proj, b_proj, Wq1, bq1)
    ones = jnp.ones((128, D), jnp.float32)
    zagg0 = jnp.zeros((N1 // NS, D), jnp.float32)
    e0 = jnp.stack([edge0_src, edge0_dst])
    agg1, cnt1 = _edge_agg0(m1, e0, zagg0, ones)
    h1, m2 = _stage_c(h[:N1], agg1, cnt1, Ww1, bw1, Wq2, bq2)
    zagg1 = jnp.zeros((N2 // NS, D), jnp.float32)
    e1 = jnp.stack([edge1_src, edge1_dst])
    agg2, cnt2 = _edge_agg1(m2, e1, zagg1, ones)
    return _stage_d(h[:N2], h1[:N2], agg2, cnt2, Ww2, bw2)


# speedup vs baseline: 9.9522x; 1.1948x over previous
import functools

import jax
import jax.numpy as jnp
from jax import lax
from jax.experimental import pallas as pl
from jax.experimental.pallas import tpu as pltpu
from jax.experimental.pallas import tpu_sc as plsc

N0, N1, N2 = 10000, 4096, 1024
E0, E1 = 320000, 32768
D = 128

NC, NS = 2, 16
NW = NC * NS


def _make_edge_agg(n_edges, n_dst, chunk=128):
    cpt = n_edges // (chunk * NW)
    nrem = (n_edges - cpt * chunk * NW) // chunk
    assert n_edges == (cpt * NW + nrem) * chunk
    assert nrem <= NW and chunk % 8 == 0 and chunk <= 128
    zr = n_dst // NS
    mesh = plsc.VectorSubcoreMesh(core_axis_name="c", subcore_axis_name="s")

    @functools.partial(
        pl.kernel,
        out_type=jax.ShapeDtypeStruct((NC, n_dst, D), jnp.float32),
        mesh=mesh,
        scratch_types=[
            pltpu.VMEM((2, chunk), jnp.int32),
            pltpu.VMEM((2, chunk), jnp.int32),
            pltpu.VMEM((chunk, D), jnp.float32),
            pltpu.VMEM((chunk, D), jnp.float32),
            pltpu.VMEM_SHARED((n_dst, D), jnp.float32),
            pltpu.SemaphoreType.DMA,
            pltpu.SemaphoreType.DMA,
            pltpu.SemaphoreType.DMA,
            pltpu.SemaphoreType.DMA,
        ],
    )
    def edge_agg(m_hbm, e_hbm, zagg_hbm, agg_out,
                 ev0, ev1, buf0, buf1, agg_sh, gsem0, gsem1, asem0, asem1):
        cid = lax.axis_index("c")
        sid = lax.axis_index("s")
        wid = cid * NS + sid

        pltpu.sync_copy(zagg_hbm, agg_sh.at[pl.ds(sid * zr, zr)])
        plsc.subcore_barrier()

        base = wid * (cpt * chunk)

        def start(off, ev, bf, gsem):
            pltpu.sync_copy(e_hbm.at[:, pl.ds(off, chunk)], ev)
            pltpu.async_copy(m_hbm.at[ev.at[0]], bf, gsem)

        def finish_gather(ev, bf, gsem, asem):
            pltpu.make_async_copy(m_hbm.at[ev.at[0]], bf, gsem).wait()
            pltpu.async_copy(bf, agg_sh.at[ev.at[1]], asem, add=True)

        def drain(ev, bf, asem):
            pltpu.make_async_copy(bf, agg_sh.at[ev.at[1]], asem).wait()

        start(base, ev0, buf0, gsem0)

        @pl.loop(0, (cpt + 1) // 2)
        def _(i):
            c = 2 * i

            @pl.when(c + 1 < cpt)
            def _():
                start(base + (c + 1) * chunk, ev1, buf1, gsem1)

            finish_gather(ev0, buf0, gsem0, asem0)

            @pl.when(c + 1 < cpt)
            def _():
                finish_gather(ev1, buf1, gsem1, asem1)

            drain(ev0, buf0, asem0)

            @pl.when(c + 2 < cpt)
            def _():
                start(base + (c + 2) * chunk, ev0, buf0, gsem0)

            @pl.when(c + 1 < cpt)
            def _():
                drain(ev1, buf1, asem1)

        if nrem:
            @pl.when(wid < nrem)
            def _():
                off = cpt * chunk * NW + wid * chunk
                start(off, ev0, buf0, gsem0)
                finish_gather(ev0, buf0, gsem0, asem0)
                drain(ev0, buf0, asem0)

        plsc.subcore_barrier()
        pltpu.sync_copy(agg_sh.at[pl.ds(sid * zr, zr)],
                        agg_out.at[cid, pl.ds(sid * zr, zr)])

    return edge_agg


def _make_hist(n_edges, n_dst, eb):
    nh = n_dst // 128
    nblk = n_edges // eb
    assert n_edges == nblk * eb

    def body(d_ref, out_ref):
        i = pl.program_id(0)
        d = d_ref[...][0]
        hi = lax.shift_right_logical(d, 7)
        lo = lax.bitwise_and(d, 127)
        hiota = lax.broadcasted_iota(jnp.int32, (nh, eb), 0)
        liota = lax.broadcasted_iota(jnp.int32, (128, eb), 0)
        a = (hiota == hi).astype(jnp.bfloat16)
        b = (liota == lo).astype(jnp.bfloat16)
        blk = jax.lax.dot_general(a, b, (((1,), (1,)), ((), ())),
                                  preferred_element_type=jnp.float32)

        @pl.when(i == 0)
        def _():
            out_ref[...] = blk

        @pl.when(i > 0)
        def _():
            out_ref[...] += blk

    def hist(dst):
        d3 = dst.reshape(nblk, 1, eb)
        return pl.pallas_call(
            body,
            grid=(nblk,),
            in_specs=[pl.BlockSpec((1, 1, eb), lambda i: (i, 0, 0))],
            out_specs=pl.BlockSpec((nh, 128), lambda i: (0, 0)),
            out_shape=jax.ShapeDtypeStruct((nh, 128), jnp.float32),
        )(d3)

    return hist


def _dot(a, b):
    return jnp.dot(a, b, preferred_element_type=jnp.float32,
                   precision=lax.Precision.HIGHEST)


def _stage_a_body(x_ref, wp_ref, bp_ref, wq_ref, bq_ref, h_ref, m_ref):
    xb = x_ref[...]
    hb = _dot(xb, wp_ref[...]) + bp_ref[...]
    h_ref[...] = hb
    m_ref[...] = jnp.maximum(_dot(hb, wq_ref[...]) + bq_ref[...], 0.0)


def _stage_a(x, W_proj, b_proj, Wq1, bq1):
    n = x.shape[0]
    blk = 1000
    grid = n // blk
    full = lambda i: (0, 0)
    return pl.pallas_call(
        _stage_a_body,
        grid=(grid,),
        in_specs=[
            pl.BlockSpec((blk, D), lambda i: (i, 0)),
            pl.BlockSpec((D, D), full),
            pl.BlockSpec((1, D), full),
            pl.BlockSpec((D, D), full),
            pl.BlockSpec((1, D), full),
        ],
        out_specs=[
            pl.BlockSpec((blk, D), lambda i: (i, 0)),
            pl.BlockSpec((blk, D), lambda i: (i, 0)),
        ],
        out_shape=[
            jax.ShapeDtypeStruct((n, D), jnp.float32),
            jax.ShapeDtypeStruct((n, D), jnp.float32),
        ],
    )(x, W_proj, b_proj.reshape(1, D), Wq1, bq1.reshape(1, D))


def _mean_from_parts(agg_ref, cnt_ref):
    p = agg_ref[...]
    cnt = cnt_ref[...][0, 0]
    return (p[0] + p[1]) / jnp.maximum(cnt, 1.0)[:, None]


def _l2norm(z):
    zn = jnp.sqrt(jnp.sum(z * z, axis=1, keepdims=True))
    return z / jnp.maximum(zn, 1e-12)


def _stage_c_body(h_ref, agg_ref, cnt_ref, wa_ref, wb_ref, bw_ref,
                  wq_ref, bq_ref, h1_ref, m2_ref):
    mean = _mean_from_parts(agg_ref, cnt_ref)
    z = _dot(h_ref[...], wa_ref[...]) + _dot(mean, wb_ref[...]) + bw_ref[...]
    h1 = _l2norm(jnp.maximum(z, 0.0))
    h1_ref[...] = h1
    m2_ref[...] = jnp.maximum(_dot(h1, wq_ref[...]) + bq_ref[...], 0.0)


def _stage_c(h4096, agg_parts, cnt_parts, Ww1, bw1, Wq2, bq2):
    blk = 512
    grid = N1 // blk
    full = lambda i: (0, 0)
    return pl.pallas_call(
        _stage_c_body,
        grid=(grid,),
        in_specs=[
            pl.BlockSpec((blk, D), lambda i: (i, 0)),
            pl.BlockSpec((NC, blk, D), lambda i: (0, i, 0)),
            pl.BlockSpec((1, 1, blk), lambda i: (i, 0, 0)),
            pl.BlockSpec((D, D), full),
            pl.BlockSpec((D, D), full),
            pl.BlockSpec((1, D), full),
            pl.BlockSpec((D, D), full),
            pl.BlockSpec((1, D), full),
        ],
        out_specs=[
            pl.BlockSpec((blk, D), lambda i: (i, 0)),
            pl.BlockSpec((blk, D), lambda i: (i, 0)),
        ],
        out_shape=[
            jax.ShapeDtypeStruct((N1, D), jnp.float32),
            jax.ShapeDtypeStruct((N1, D), jnp.float32),
        ],
    )(h4096, agg_parts, cnt_parts, Ww1[:D], Ww1[D:], bw1.reshape(1, D),
      Wq2, bq2.reshape(1, D))


def _stage_d_body(h_ref, h1_ref, agg_ref, cnt_ref, wa_ref, wb_ref, bw_ref,
                  out_ref):
    mean = _mean_from_parts(agg_ref, cnt_ref)
    z = _dot(h1_ref[...], wa_ref[...]) + _dot(mean, wb_ref[...]) + bw_ref[...]
    out_ref[...] = h_ref[...] + _l2norm(jnp.maximum(z, 0.0))


def _stage_d(h1024, h1_1024, agg_parts, cnt_parts, Ww2, bw2):
    blk = 512
    grid = N2 // blk
    full = lambda i: (0, 0)
    return pl.pallas_call(
        _stage_d_body,
        grid=(grid,),
        in_specs=[
            pl.BlockSpec((blk, D), lambda i: (i, 0)),
            pl.BlockSpec((blk, D), lambda i: (i, 0)),
            pl.BlockSpec((NC, blk, D), lambda i: (0, i, 0)),
            pl.BlockSpec((1, 1, blk), lambda i: (i, 0, 0)),
            pl.BlockSpec((D, D), full),
            pl.BlockSpec((D, D), full),
            pl.BlockSpec((1, D), full),
        ],
        out_specs=pl.BlockSpec((blk, D), lambda i: (i, 0)),
        out_shape=jax.ShapeDtypeStruct((N2, D), jnp.float32),
    )(h1024, h1_1024, agg_parts, cnt_parts, Ww2[:D], Ww2[D:],
      bw2.reshape(1, D))


_edge_agg0 = _make_edge_agg(E0, N1)
_edge_agg1 = _make_edge_agg(E1, N2)
_hist0 = _make_hist(E0, N1, 2500)
_hist1 = _make_hist(E1, N2, 2048)


def kernel(x, edge0_src, edge0_dst, edge1_src, edge1_dst,
           W_proj, b_proj, Wq1, bq1, Ww1, bw1, Wq2, bq2, Ww2, bw2):
    h, m1 = _stage_a(x, W_proj, b_proj, Wq1, bq1)
    zagg0 = jnp.zeros((N1 // NS, D), jnp.float32)
    e0 = jnp.stack([edge0_src, edge0_dst])
    agg1 = _edge_agg0(m1, e0, zagg0)
    cnt1 = _hist0(edge0_dst).reshape(N1 // 512, 1, 512)
    h1, m2 = _stage_c(h[:N1], agg1, cnt1, Ww1, bw1, Wq2, bq2)
    zagg1 = jnp.zeros((N2 // NS, D), jnp.float32)
    e1 = jnp.stack([edge1_src, edge1_dst])
    agg2 = _edge_agg1(m2, e1, zagg1)
    cnt2 = _hist1(edge1_dst).reshape(N2 // 512, 1, 512)
    return _stage_d(h[:N2], h1[:N2], agg2, cnt2, Ww2, bw2)


# speedup vs baseline: 10.7799x; 1.0832x over previous
import functools

import jax
import jax.numpy as jnp
from jax import lax
from jax.experimental import pallas as pl
from jax.experimental.pallas import tpu as pltpu
from jax.experimental.pallas import tpu_sc as plsc

N0, N1, N2 = 10000, 4096, 1024
E0, E1 = 320000, 32768
D = 128

NC, NS = 2, 16
NW = NC * NS


def _make_edge_agg(n_edges, n_dst, chunk=128):
    cpt = n_edges // (chunk * NW)
    nrem = (n_edges - cpt * chunk * NW) // chunk
    assert n_edges == (cpt * NW + nrem) * chunk
    assert nrem <= NW and chunk % 8 == 0 and chunk <= 128
    zr = n_dst // NS
    mesh = plsc.VectorSubcoreMesh(core_axis_name="c", subcore_axis_name="s")

    @functools.partial(
        pl.kernel,
        out_type=jax.ShapeDtypeStruct((NC, n_dst, D), jnp.float32),
        mesh=mesh,
        scratch_types=[
            pltpu.VMEM((2, chunk), jnp.int32),
            pltpu.VMEM((2, chunk), jnp.int32),
            pltpu.VMEM((chunk, D), jnp.float32),
            pltpu.VMEM((chunk, D), jnp.float32),
            pltpu.VMEM_SHARED((n_dst, D), jnp.float32),
            pltpu.SemaphoreType.DMA,
            pltpu.SemaphoreType.DMA,
            pltpu.SemaphoreType.DMA,
            pltpu.SemaphoreType.DMA,
        ],
    )
    def edge_agg(m_hbm, e_hbm, zagg_hbm, agg_out,
                 ev0, ev1, buf0, buf1, agg_sh, gsem0, gsem1, asem0, asem1):
        cid = lax.axis_index("c")
        sid = lax.axis_index("s")
        wid = cid * NS + sid

        pltpu.sync_copy(zagg_hbm, agg_sh.at[pl.ds(sid * zr, zr)])
        plsc.subcore_barrier()

        base = wid * (cpt * chunk)

        def start(off, ev, bf, gsem):
            pltpu.sync_copy(e_hbm.at[:, pl.ds(off, chunk)], ev)
            pltpu.async_copy(m_hbm.at[ev.at[0]], bf, gsem)

        def finish_gather(ev, bf, gsem, asem):
            pltpu.make_async_copy(m_hbm.at[ev.at[0]], bf, gsem).wait()
            pltpu.async_copy(bf, agg_sh.at[ev.at[1]], asem, add=True)

        def drain(ev, bf, asem):
            pltpu.make_async_copy(bf, agg_sh.at[ev.at[1]], asem).wait()

        start(base, ev0, buf0, gsem0)

        @pl.loop(0, (cpt + 1) // 2)
        def _(i):
            c = 2 * i

            @pl.when(c + 1 < cpt)
            def _():
                start(base + (c + 1) * chunk, ev1, buf1, gsem1)

            finish_gather(ev0, buf0, gsem0, asem0)

            @pl.when(c + 1 < cpt)
            def _():
                finish_gather(ev1, buf1, gsem1, asem1)

            drain(ev0, buf0, asem0)

            @pl.when(c + 2 < cpt)
            def _():
                start(base + (c + 2) * chunk, ev0, buf0, gsem0)

            @pl.when(c + 1 < cpt)
            def _():
                drain(ev1, buf1, asem1)

        if nrem:
            @pl.when(wid < nrem)
            def _():
                off = cpt * chunk * NW + wid * chunk
                start(off, ev0, buf0, gsem0)
                finish_gather(ev0, buf0, gsem0, asem0)
                drain(ev0, buf0, asem0)

        plsc.subcore_barrier()
        pltpu.sync_copy(agg_sh.at[pl.ds(sid * zr, zr)],
                        agg_out.at[cid, pl.ds(sid * zr, zr)])

    return edge_agg


def _make_hist(n_edges, n_dst, eb):
    nh = n_dst // 128
    nblk = n_edges // eb
    assert n_edges == nblk * eb

    def body(d_ref, out_ref):
        i = pl.program_id(0)
        d = d_ref[...][0]
        hi = lax.shift_right_logical(d, 7)
        lo = lax.bitwise_and(d, 127)
        hiota = lax.broadcasted_iota(jnp.int32, (nh, eb), 0)
        liota = lax.broadcasted_iota(jnp.int32, (128, eb), 0)
        a = (hiota == hi).astype(jnp.bfloat16)
        b = (liota == lo).astype(jnp.bfloat16)
        blk = jax.lax.dot_general(a, b, (((1,), (1,)), ((), ())),
                                  preferred_element_type=jnp.float32)

        @pl.when(i == 0)
        def _():
            out_ref[...] = blk

        @pl.when(i > 0)
        def _():
            out_ref[...] += blk

    def hist(dst):
        d3 = dst.reshape(nblk, 1, eb)
        return pl.pallas_call(
            body,
            grid=(nblk,),
            in_specs=[pl.BlockSpec((1, 1, eb), lambda i: (i, 0, 0))],
            out_specs=pl.BlockSpec((nh, 128), lambda i: (0, 0)),
            out_shape=jax.ShapeDtypeStruct((nh, 128), jnp.float32),
        )(d3)

    return hist


def _dot(a, b):
    return jnp.dot(a, b, preferred_element_type=jnp.float32,
                   precision=lax.Precision.HIGHEST)


def _stage_m1_body(x_ref, wp_ref, bp_ref, wq_ref, bq_ref, m_ref):
    w2 = _dot(wp_ref[...], wq_ref[...])
    b2 = _dot(bp_ref[...], wq_ref[...]) + bq_ref[...]
    m_ref[...] = jnp.maximum(_dot(x_ref[...], w2) + b2, 0.0)


def _stage_m1(x, W_proj, b_proj, Wq1, bq1):
    n = x.shape[0]
    blk = 1000
    grid = n // blk
    full = lambda i: (0, 0)
    return pl.pallas_call(
        _stage_m1_body,
        grid=(grid,),
        in_specs=[
            pl.BlockSpec((blk, D), lambda i: (i, 0)),
            pl.BlockSpec((D, D), full),
            pl.BlockSpec((1, D), full),
            pl.BlockSpec((D, D), full),
            pl.BlockSpec((1, D), full),
        ],
        out_specs=pl.BlockSpec((blk, D), lambda i: (i, 0)),
        out_shape=jax.ShapeDtypeStruct((n, D), jnp.float32),
    )(x, W_proj, b_proj.reshape(1, D), Wq1, bq1.reshape(1, D))


def _stage_h_body(x_ref, wp_ref, bp_ref, h_ref):
    h_ref[...] = _dot(x_ref[...], wp_ref[...]) + bp_ref[...]


def _stage_h(x4096, W_proj, b_proj):
    blk = 512
    grid = N1 // blk
    full = lambda i: (0, 0)
    return pl.pallas_call(
        _stage_h_body,
        grid=(grid,),
        in_specs=[
            pl.BlockSpec((blk, D), lambda i: (i, 0)),
            pl.BlockSpec((D, D), full),
            pl.BlockSpec((1, D), full),
        ],
        out_specs=pl.BlockSpec((blk, D), lambda i: (i, 0)),
        out_shape=jax.ShapeDtypeStruct((N1, D), jnp.float32),
    )(x4096, W_proj, b_proj.reshape(1, D))


def _mean_from_parts(agg_ref, cnt_ref):
    p = agg_ref[...]
    cnt = cnt_ref[...][0, 0]
    return (p[0] + p[1]) / jnp.maximum(cnt, 1.0)[:, None]


def _l2norm(z):
    zn = jnp.sqrt(jnp.sum(z * z, axis=1, keepdims=True))
    return z / jnp.maximum(zn, 1e-12)


def _stage_c_body(h_ref, agg_ref, cnt_ref, wa_ref, wb_ref, bw_ref,
                  wq_ref, bq_ref, h1_ref, m2_ref):
    mean = _mean_from_parts(agg_ref, cnt_ref)
    z = _dot(h_ref[...], wa_ref[...]) + _dot(mean, wb_ref[...]) + bw_ref[...]
    h1 = _l2norm(jnp.maximum(z, 0.0))
    h1_ref[...] = h1
    m2_ref[...] = jnp.maximum(_dot(h1, wq_ref[...]) + bq_ref[...], 0.0)


def _stage_c(h4096, agg_parts, cnt_parts, Ww1, bw1, Wq2, bq2):
    blk = 512
    grid = N1 // blk
    full = lambda i: (0, 0)
    return pl.pallas_call(
        _stage_c_body,
        grid=(grid,),
        in_specs=[
            pl.BlockSpec((blk, D), lambda i: (i, 0)),
            pl.BlockSpec((NC, blk, D), lambda i: (0, i, 0)),
            pl.BlockSpec((1, 1, blk), lambda i: (i, 0, 0)),
            pl.BlockSpec((D, D), full),
            pl.BlockSpec((D, D), full),
            pl.BlockSpec((1, D), full),
            pl.BlockSpec((D, D), full),
            pl.BlockSpec((1, D), full),
        ],
        out_specs=[
            pl.BlockSpec((blk, D), lambda i: (i, 0)),
            pl.BlockSpec((blk, D), lambda i: (i, 0)),
        ],
        out_shape=[
            jax.ShapeDtypeStruct((N1, D), jnp.float32),
            jax.ShapeDtypeStruct((N1, D), jnp.float32),
        ],
    )(h4096, agg_parts, cnt_parts, Ww1[:D], Ww1[D:], bw1.reshape(1, D),
      Wq2, bq2.reshape(1, D))


def _stage_d_body(h_ref, h1_ref, agg_ref, cnt_ref, wa_ref, wb_ref, bw_ref,
                  out_ref):
    mean = _mean_from_parts(agg_ref, cnt_ref)
    z = _dot(h1_ref[...], wa_ref[...]) + _dot(mean, wb_ref[...]) + bw_ref[...]
    out_ref[...] = h_ref[...] + _l2norm(jnp.maximum(z, 0.0))


def _stage_d(h1024, h1_1024, agg_parts, cnt_parts, Ww2, bw2):
    blk = 512
    grid = N2 // blk
    full = lambda i: (0, 0)
    return pl.pallas_call(
        _stage_d_body,
        grid=(grid,),
        in_specs=[
            pl.BlockSpec((blk, D), lambda i: (i, 0)),
            pl.BlockSpec((blk, D), lambda i: (i, 0)),
            pl.BlockSpec((NC, blk, D), lambda i: (0, i, 0)),
            pl.BlockSpec((1, 1, blk), lambda i: (i, 0, 0)),
            pl.BlockSpec((D, D), full),
            pl.BlockSpec((D, D), full),
            pl.BlockSpec((1, D), full),
        ],
        out_specs=pl.BlockSpec((blk, D), lambda i: (i, 0)),
        out_shape=jax.ShapeDtypeStruct((N2, D), jnp.float32),
    )(h1024, h1_1024, agg_parts, cnt_parts, Ww2[:D], Ww2[D:],
      bw2.reshape(1, D))


_edge_agg0 = _make_edge_agg(E0, N1)
_edge_agg1 = _make_edge_agg(E1, N2)
_hist0 = _make_hist(E0, N1, 20000)
_hist1 = _make_hist(E1, N2, 8192)


def kernel(x, edge0_src, edge0_dst, edge1_src, edge1_dst,
           W_proj, b_proj, Wq1, bq1, Ww1, bw1, Wq2, bq2, Ww2, bw2):
    m1 = _stage_m1(x, W_proj, b_proj, Wq1, bq1)
    zagg0 = jnp.zeros((N1 // NS, D), jnp.float32)
    e0 = jnp.stack([edge0_src, edge0_dst])
    agg1 = _edge_agg0(m1, e0, zagg0)
    h = _stage_h(x[:N1], W_proj, b_proj)
    cnt1 = _hist0(edge0_dst).reshape(N1 // 512, 1, 512)
    h1, m2 = _stage_c(h, agg1, cnt1, Ww1, bw1, Wq2, bq2)
    zagg1 = jnp.zeros((N2 // NS, D), jnp.float32)
    e1 = jnp.stack([edge1_src, edge1_dst])
    agg2 = _edge_agg1(m2, e1, zagg1)
    cnt2 = _hist1(edge1_dst).reshape(N2 // 512, 1, 512)
    return _stage_d(h[:N2], h1[:N2], agg2, cnt2, Ww2, bw2)
